# baseline (speedup 1.0000x reference)
import functools
import math

import jax
import jax.numpy as jnp
from jax.experimental import pallas as pl
from jax.experimental.pallas import tpu as pltpu

_INV_SQRT2 = 1.0 / math.sqrt(2.0)


def _router_block(x_ref, w1_ref, b1_ref, w2_ref, b2c_ref,
                  packets_ref, probs_ref):
    h = jnp.dot(x_ref[...], w1_ref[...], preferred_element_type=jnp.float32)
    h = h + b1_ref[...]
    h = 0.5 * h * (1.0 + jax.lax.erf(h * _INV_SQRT2))
    logits_t = jax.lax.dot_general(
        w2_ref[...], h, (((0,), (1,)), ((), ())),
        preferred_element_type=jnp.float32)
    logits_t = logits_t + b2c_ref[...]
    pcount = logits_t.shape[0]
    m = jnp.max(logits_t, axis=0, keepdims=True)
    row_idx = jax.lax.broadcasted_iota(jnp.int32, logits_t.shape, 0)
    cand = jnp.where(logits_t == m, row_idx, pcount)
    packets_ref[...] = jnp.min(cand, axis=0, keepdims=True)
    e = jnp.exp(logits_t - m)
    probs_ref[...] = e / jnp.sum(e, axis=0, keepdims=True)


@functools.partial(jax.jit, static_argnames=("block_n",))
def kernel(x, W1, b1, W2, b2, block_n: int = 8192):
    n, d = x.shape
    h_dim = W1.shape[1]
    p = W2.shape[1]
    b2c = b2.reshape(p, 1)
    packets_row, probs_t = pl.pallas_call(
        _router_block,
        grid=(n // block_n,),
        in_specs=[
            pl.BlockSpec((block_n, d), lambda i: (i, 0)),
            pl.BlockSpec((d, h_dim), lambda i: (0, 0)),
            pl.BlockSpec((h_dim,), lambda i: (0,)),
            pl.BlockSpec((h_dim, p), lambda i: (0, 0)),
            pl.BlockSpec((p, 1), lambda i: (0, 0)),
        ],
        out_specs=[
            pl.BlockSpec((1, block_n), lambda i: (0, i)),
            pl.BlockSpec((p, block_n), lambda i: (0, i)),
        ],
        out_shape=[
            jax.ShapeDtypeStruct((1, n), jnp.int32),
            jax.ShapeDtypeStruct((p, n), jnp.float32),
        ],
        compiler_params=pltpu.CompilerParams(
            dimension_semantics=("arbitrary",),
        ),
    )(x, W1, b1, W2, b2c)
    return packets_row.reshape(n), probs_t.T


# speedup vs baseline: 1.0071x; 1.0071x over previous
import functools
import math

import jax
import jax.numpy as jnp
from jax.experimental import pallas as pl
from jax.experimental.pallas import tpu as pltpu

_INV_SQRT2 = 1.0 / math.sqrt(2.0)


def _router_kernel(block_n, nbuf, nsteps, x_hbm, w1_ref, b1_ref, w2_ref,
                   b2c_ref, packets_ref, probs_ref, xbuf, dma_sems):
    i = pl.program_id(0)

    def start_copy(step, slot):
        pltpu.make_async_copy(
            x_hbm.at[pl.ds(step * block_n, block_n), :],
            xbuf.at[slot],
            dma_sems.at[slot],
        ).start()

    @pl.when(i == 0)
    def _():
        for s in range(min(nbuf - 1, nsteps)):
            start_copy(s, s)

    refill = i + nbuf - 1

    @pl.when(refill < nsteps)
    def _():
        start_copy(jnp.minimum(refill, nsteps - 1), refill % nbuf)

    slot = jax.lax.rem(i, nbuf)
    pltpu.make_async_copy(
        x_hbm.at[pl.ds(i * block_n, block_n), :],
        xbuf.at[slot],
        dma_sems.at[slot],
    ).wait()

    h = jnp.dot(xbuf[slot], w1_ref[...], preferred_element_type=jnp.float32)
    h = h + b1_ref[...]
    h = 0.5 * h * (1.0 + jax.lax.erf(h * _INV_SQRT2))
    logits_t = jax.lax.dot_general(
        w2_ref[...], h, (((0,), (1,)), ((), ())),
        preferred_element_type=jnp.float32)
    logits_t = logits_t + b2c_ref[...]
    pcount = logits_t.shape[0]
    m = jnp.max(logits_t, axis=0, keepdims=True)
    row_idx = jax.lax.broadcasted_iota(jnp.int32, logits_t.shape, 0)
    cand = jnp.where(logits_t == m, row_idx, pcount)
    packets_ref[...] = jnp.min(cand, axis=0, keepdims=True)
    e = jnp.exp(logits_t - m)
    probs_ref[...] = e / jnp.sum(e, axis=0, keepdims=True)


@functools.partial(jax.jit, static_argnames=("block_n", "nbuf"))
def kernel(x, W1, b1, W2, b2, block_n: int = 4096, nbuf: int = 3):
    n, d = x.shape
    h_dim = W1.shape[1]
    p = W2.shape[1]
    nsteps = n // block_n
    b2c = b2.reshape(p, 1)
    packets_row, probs_t = pl.pallas_call(
        functools.partial(_router_kernel, block_n, nbuf, nsteps),
        grid=(nsteps,),
        in_specs=[
            pl.BlockSpec(memory_space=pltpu.MemorySpace.HBM),
            pl.BlockSpec((d, h_dim), lambda i: (0, 0)),
            pl.BlockSpec((h_dim,), lambda i: (0,)),
            pl.BlockSpec((h_dim, p), lambda i: (0, 0)),
            pl.BlockSpec((p, 1), lambda i: (0, 0)),
        ],
        out_specs=[
            pl.BlockSpec((1, block_n), lambda i: (0, i)),
            pl.BlockSpec((p, block_n), lambda i: (0, i)),
        ],
        out_shape=[
            jax.ShapeDtypeStruct((1, n), jnp.int32),
            jax.ShapeDtypeStruct((p, n), jnp.float32),
        ],
        scratch_shapes=[
            pltpu.VMEM((nbuf, block_n, d), jnp.float32),
            pltpu.SemaphoreType.DMA((nbuf,)),
        ],
        compiler_params=pltpu.CompilerParams(
            dimension_semantics=("arbitrary",),
        ),
    )(x, W1, b1, W2, b2c)
    return packets_row.reshape(n), probs_t.T


# speedup vs baseline: 1.0515x; 1.0441x over previous
import functools
import math

import jax
import jax.numpy as jnp
from jax.experimental import pallas as pl
from jax.experimental.pallas import tpu as pltpu

_INV_SQRT2 = 1.0 / math.sqrt(2.0)


def _router_block(x_ref, w1_ref, b1_ref, w2_ref, b2c_ref,
                  packets_ref, probs_ref):
    h = jnp.dot(x_ref[...], w1_ref[...], preferred_element_type=jnp.float32)
    h = h + b1_ref[...]
    h = 0.5 * h * (1.0 + jax.lax.erf(h * _INV_SQRT2))
    logits_t = jax.lax.dot_general(
        w2_ref[...], h, (((0,), (1,)), ((), ())),
        preferred_element_type=jnp.float32)
    logits_t = logits_t + b2c_ref[...]
    pcount = logits_t.shape[0]
    m = jnp.max(logits_t, axis=0, keepdims=True)
    row_idx = jax.lax.broadcasted_iota(jnp.int32, logits_t.shape, 0)
    cand = jnp.where(logits_t == m, row_idx, pcount)
    packets_ref[...] = jnp.min(cand, axis=0, keepdims=True)
    e = jnp.exp(logits_t - m)
    probs_ref[...] = e / jnp.sum(e, axis=0, keepdims=True)


@functools.partial(jax.jit, static_argnames=("block_n",))
def kernel(x, W1, b1, W2, b2, block_n: int = 4096):
    n, d = x.shape
    h_dim = W1.shape[1]
    p = W2.shape[1]
    b2c = b2.reshape(p, 1)
    packets_row, probs_t = pl.pallas_call(
        _router_block,
        grid=(n // block_n,),
        in_specs=[
            pl.BlockSpec((block_n, d), lambda i: (i, 0)),
            pl.BlockSpec((d, h_dim), lambda i: (0, 0)),
            pl.BlockSpec((h_dim,), lambda i: (0,)),
            pl.BlockSpec((h_dim, p), lambda i: (0, 0)),
            pl.BlockSpec((p, 1), lambda i: (0, 0)),
        ],
        out_specs=[
            pl.BlockSpec((1, block_n), lambda i: (0, i)),
            pl.BlockSpec((p, block_n), lambda i: (0, i)),
        ],
        out_shape=[
            jax.ShapeDtypeStruct((1, n), jnp.int32),
            jax.ShapeDtypeStruct((p, n), jnp.float32),
        ],
        compiler_params=pltpu.CompilerParams(
            dimension_semantics=("parallel",),
        ),
    )(x, W1, b1, W2, b2c)
    return packets_row.reshape(n), probs_t.T
